# Initial kernel scaffold; baseline (speedup 1.0000x reference)
import functools

import jax
import jax.numpy as jnp
from jax import lax
from jax.experimental import pallas as pl
from jax.experimental.pallas import tpu as pltpu
from jax.experimental.pallas import tpu_sc as plsc

N_NODES = 10000
N_EDGES = 320000
NC = 2
NS = 16
NW = NC * NS
EDGES_PER_WORKER = N_EDGES // NW
CHUNK = 80
NCHUNKS = EDGES_PER_WORKER // CHUNK


def _segment_sum_sc(x, src, dst):
    n, d = x.shape
    mesh = plsc.VectorSubcoreMesh(core_axis_name="c", subcore_axis_name="s")
    zeros = jnp.zeros((n, d), jnp.float32)

    @functools.partial(
        pl.kernel,
        out_type=jax.ShapeDtypeStruct((NC, n, d), jnp.float32),
        mesh=mesh,
        scratch_types=[
            pltpu.VMEM((CHUNK,), jnp.int32),
            pltpu.VMEM((CHUNK,), jnp.int32),
            pltpu.VMEM((CHUNK, d), jnp.float32),
            pltpu.VMEM_SHARED((n, d), jnp.float32),
            pltpu.SemaphoreType.DMA,
        ],
    )
    def seg_sum(x_hbm, src_hbm, dst_hbm, zeros_hbm, out_hbm,
                sidx, didx, rows, acc, sem):
        cid = lax.axis_index("c")
        sid = lax.axis_index("s")
        wid = sid * NC + cid

        @pl.when(sid == 0)
        def _():
            pltpu.sync_copy(zeros_hbm, acc)

        plsc.subcore_barrier()

        base = wid * EDGES_PER_WORKER

        @pl.loop(0, NCHUNKS)
        def _(i):
            off = base + i * CHUNK
            pltpu.sync_copy(src_hbm.at[pl.ds(off, CHUNK)], sidx)
            pltpu.sync_copy(dst_hbm.at[pl.ds(off, CHUNK)], didx)
            pltpu.async_copy(x_hbm.at[sidx], rows, sem).wait()
            pltpu.sync_copy(rows, acc.at[didx], add=True)

        plsc.subcore_barrier()

        @pl.when(sid == 0)
        def _():
            pltpu.sync_copy(acc, out_hbm.at[cid])

    return seg_sum(x, src, dst, zeros)


def _mlp1_tc(x, p0, p1, W1, b1, W2, b2, eps0):
    n, d_in = x.shape
    h_dim = W1.shape[1]
    blk = 2000

    def body(eps_ref, x_ref, p0_ref, p1_ref, w1_ref, b1_ref, w2_ref, b2_ref,
             o_ref):
        t = (1.0 + eps_ref[0]) * x_ref[...] + p0_ref[...] + p1_ref[...]
        h = jnp.dot(t, w1_ref[...], preferred_element_type=jnp.float32,
                    precision=lax.Precision.HIGHEST) + b1_ref[...]
        h = jnp.maximum(h, 0.0)
        h = jnp.dot(h, w2_ref[...], preferred_element_type=jnp.float32,
                    precision=lax.Precision.HIGHEST) + b2_ref[...]
        o_ref[...] = jnp.maximum(h, 0.0)

    grid = (n // blk,)
    row_spec = pl.BlockSpec((blk, d_in), lambda i: (i, 0))
    return pl.pallas_call(
        body,
        grid=grid,
        in_specs=[
            pl.BlockSpec(memory_space=pltpu.SMEM),
            row_spec, row_spec, row_spec,
            pl.BlockSpec((d_in, h_dim), lambda i: (0, 0)),
            pl.BlockSpec((1, h_dim), lambda i: (0, 0)),
            pl.BlockSpec((h_dim, h_dim), lambda i: (0, 0)),
            pl.BlockSpec((1, h_dim), lambda i: (0, 0)),
        ],
        out_specs=pl.BlockSpec((blk, h_dim), lambda i: (i, 0)),
        out_shape=jax.ShapeDtypeStruct((n, h_dim), jnp.float32),
    )(eps0.reshape(1), x, p0, p1, W1, b1.reshape(1, -1), W2, b2.reshape(1, -1))


def _mlp2_tc(h, p0, p1, W3, b3, W4, b4, eps1):
    n, h_dim = h.shape
    d_out = W4.shape[1]
    blk = 2000

    def body(eps_ref, h_ref, p0_ref, p1_ref, w3_ref, b3_ref, w4_ref, b4_ref,
             o_ref):
        t = (1.0 + eps_ref[0]) * h_ref[...] + p0_ref[...] + p1_ref[...]
        g = jnp.dot(t, w3_ref[...], preferred_element_type=jnp.float32,
                    precision=lax.Precision.HIGHEST) + b3_ref[...]
        g = jnp.maximum(g, 0.0)
        logits = jnp.dot(g, w4_ref[...], preferred_element_type=jnp.float32,
                         precision=lax.Precision.HIGHEST) + b4_ref[...]
        m = jnp.max(logits, axis=1, keepdims=True)
        z = logits - m
        lse = jnp.log(jnp.sum(jnp.exp(z), axis=1, keepdims=True))
        o_ref[...] = z - lse

    grid = (n // blk,)
    row_spec = pl.BlockSpec((blk, h_dim), lambda i: (i, 0))
    return pl.pallas_call(
        body,
        grid=grid,
        in_specs=[
            pl.BlockSpec(memory_space=pltpu.SMEM),
            row_spec, row_spec, row_spec,
            pl.BlockSpec((h_dim, h_dim), lambda i: (0, 0)),
            pl.BlockSpec((1, h_dim), lambda i: (0, 0)),
            pl.BlockSpec((h_dim, d_out), lambda i: (0, 0)),
            pl.BlockSpec((1, d_out), lambda i: (0, 0)),
        ],
        out_specs=pl.BlockSpec((blk, d_out), lambda i: (i, 0)),
        out_shape=jax.ShapeDtypeStruct((n, d_out), jnp.float32),
    )(eps1.reshape(1), h, p0, p1, W3, b3.reshape(1, -1), W4, b4.reshape(1, -1))


def kernel(x, edge_index, W1, b1, W2, b2, eps0, W3, b3, W4, b4, eps1):
    src = edge_index[0]
    dst = edge_index[1]

    parts = _segment_sum_sc(x, src, dst)
    h = _mlp1_tc(x, parts[0], parts[1], W1, b1, W2, b2, eps0)

    parts2 = _segment_sum_sc(h, src, dst)
    return _mlp2_tc(h, parts2[0], parts2[1], W3, b3, W4, b4, eps1)


# speedup vs baseline: 4.7753x; 4.7753x over previous
import functools

import jax
import jax.numpy as jnp
from jax import lax
from jax.experimental import pallas as pl
from jax.experimental.pallas import tpu as pltpu
from jax.experimental.pallas import tpu_sc as plsc

N_NODES = 10000
N_EDGES = 320000
NC = 2
NS = 16
NW = NC * NS
EDGES_PER_WORKER = N_EDGES // NW
CHUNK = 80
NCHUNKS = EDGES_PER_WORKER // CHUNK


def _segment_sum_sc(x, src, dst):
    n, d = x.shape
    mesh = plsc.VectorSubcoreMesh(core_axis_name="c", subcore_axis_name="s")
    zeros = jnp.zeros((n, d), jnp.float32)

    @functools.partial(
        pl.kernel,
        out_type=jax.ShapeDtypeStruct((NC, n, d), jnp.float32),
        mesh=mesh,
        scratch_types=[
            pltpu.VMEM((CHUNK,), jnp.int32),
            pltpu.VMEM((CHUNK,), jnp.int32),
            pltpu.VMEM((CHUNK, d), jnp.float32),
            pltpu.VMEM_SHARED((n, d), jnp.float32),
            pltpu.SemaphoreType.DMA,
        ],
    )
    def seg_sum(x_hbm, src_hbm, dst_hbm, zeros_hbm, out_hbm,
                sidx, didx, rows, acc, sem):
        cid = lax.axis_index("c")
        sid = lax.axis_index("s")
        wid = sid * NC + cid

        @pl.when(sid == 0)
        def _():
            pltpu.sync_copy(zeros_hbm, acc)

        plsc.subcore_barrier()

        base = wid * EDGES_PER_WORKER

        @pl.loop(0, NCHUNKS)
        def _(i):
            off = base + i * CHUNK
            pltpu.sync_copy(src_hbm.at[pl.ds(off, CHUNK)], sidx)
            pltpu.sync_copy(dst_hbm.at[pl.ds(off, CHUNK)], didx)
            pltpu.async_copy(x_hbm.at[sidx], rows, sem).wait()
            pltpu.sync_copy(rows, acc.at[didx], add=True)

        plsc.subcore_barrier()

        @pl.when(sid == 0)
        def _():
            pltpu.sync_copy(acc, out_hbm.at[cid])

    return seg_sum(x, src, dst, zeros)


def _mlp1_tc(x, p0, p1, W1, b1, W2, b2, eps0):
    n, d_in = x.shape
    h_dim = W1.shape[1]
    blk = 2000

    def body(eps_ref, x_ref, p0_ref, p1_ref, w1_ref, b1_ref, w2_ref, b2_ref,
             o_ref):
        t = (1.0 + eps_ref[0]) * x_ref[...] + p0_ref[...] + p1_ref[...]
        h = jnp.dot(t, w1_ref[...], preferred_element_type=jnp.float32,
                    precision=lax.Precision.HIGHEST) + b1_ref[...]
        h = jnp.maximum(h, 0.0)
        h = jnp.dot(h, w2_ref[...], preferred_element_type=jnp.float32,
                    precision=lax.Precision.HIGHEST) + b2_ref[...]
        h = jnp.maximum(h, 0.0)
        o_ref[...] = jnp.concatenate([h, jnp.zeros_like(h)], axis=1)

    grid = (n // blk,)
    row_spec = pl.BlockSpec((blk, d_in), lambda i: (i, 0))
    return pl.pallas_call(
        body,
        grid=grid,
        in_specs=[
            pl.BlockSpec(memory_space=pltpu.SMEM),
            row_spec, row_spec, row_spec,
            pl.BlockSpec((d_in, h_dim), lambda i: (0, 0)),
            pl.BlockSpec((1, h_dim), lambda i: (0, 0)),
            pl.BlockSpec((h_dim, h_dim), lambda i: (0, 0)),
            pl.BlockSpec((1, h_dim), lambda i: (0, 0)),
        ],
        out_specs=pl.BlockSpec((blk, 2 * h_dim), lambda i: (i, 0)),
        out_shape=jax.ShapeDtypeStruct((n, 2 * h_dim), jnp.float32),
    )(eps0.reshape(1), x, p0, p1, W1, b1.reshape(1, -1), W2, b2.reshape(1, -1))


def _mlp2_tc(h, p0, p1, W3, b3, W4, b4, eps1):
    n, pad_dim = h.shape
    h_dim = W3.shape[0]
    d_out = W4.shape[1]
    blk = 2000

    def body(eps_ref, h_ref, p0_ref, p1_ref, w3_ref, b3_ref, w4_ref, b4_ref,
             o_ref):
        t = (1.0 + eps_ref[0]) * h_ref[...] + p0_ref[...] + p1_ref[...]
        t = t[:, :h_dim]
        g = jnp.dot(t, w3_ref[...], preferred_element_type=jnp.float32,
                    precision=lax.Precision.HIGHEST) + b3_ref[...]
        g = jnp.maximum(g, 0.0)
        logits = jnp.dot(g, w4_ref[...], preferred_element_type=jnp.float32,
                         precision=lax.Precision.HIGHEST) + b4_ref[...]
        m = jnp.max(logits, axis=1, keepdims=True)
        z = logits - m
        lse = jnp.log(jnp.sum(jnp.exp(z), axis=1, keepdims=True))
        o_ref[...] = z - lse

    grid = (n // blk,)
    row_spec = pl.BlockSpec((blk, pad_dim), lambda i: (i, 0))
    return pl.pallas_call(
        body,
        grid=grid,
        in_specs=[
            pl.BlockSpec(memory_space=pltpu.SMEM),
            row_spec, row_spec, row_spec,
            pl.BlockSpec((h_dim, h_dim), lambda i: (0, 0)),
            pl.BlockSpec((1, h_dim), lambda i: (0, 0)),
            pl.BlockSpec((h_dim, d_out), lambda i: (0, 0)),
            pl.BlockSpec((1, d_out), lambda i: (0, 0)),
        ],
        out_specs=pl.BlockSpec((blk, d_out), lambda i: (i, 0)),
        out_shape=jax.ShapeDtypeStruct((n, d_out), jnp.float32),
    )(eps1.reshape(1), h, p0, p1, W3, b3.reshape(1, -1), W4, b4.reshape(1, -1))


def kernel(x, edge_index, W1, b1, W2, b2, eps0, W3, b3, W4, b4, eps1):
    src = edge_index[0]
    dst = edge_index[1]

    parts = _segment_sum_sc(x, src, dst)
    h = _mlp1_tc(x, parts[0], parts[1], W1, b1, W2, b2, eps0)

    parts2 = _segment_sum_sc(h, src, dst)
    return _mlp2_tc(h, parts2[0], parts2[1], W3, b3, W4, b4, eps1)


# speedup vs baseline: 10.4588x; 2.1902x over previous
import functools

import jax
import jax.numpy as jnp
from jax import lax
from jax.experimental import pallas as pl
from jax.experimental.pallas import tpu as pltpu
from jax.experimental.pallas import tpu_sc as plsc

N_NODES = 10000
N_EDGES = 320000
NC = 2
NS = 16
NW = NC * NS
EDGES_PER_WORKER = N_EDGES // NW
CHUNK = 80
NCHUNKS = EDGES_PER_WORKER // CHUNK


STRIPE = 624


def _segment_sum_sc(x, src, dst):
    n, d = x.shape
    mesh = plsc.VectorSubcoreMesh(core_axis_name="c", subcore_axis_name="s")
    zeros = jnp.zeros((n, d), jnp.float32)

    @functools.partial(
        pl.kernel,
        out_type=jax.ShapeDtypeStruct((NC, n, d), jnp.float32),
        mesh=mesh,
        scratch_types=[
            pltpu.VMEM((EDGES_PER_WORKER,), jnp.int32),
            pltpu.VMEM((CHUNK,), jnp.int32),
            pltpu.VMEM((CHUNK,), jnp.int32),
            pltpu.VMEM((CHUNK, d), jnp.float32),
            pltpu.VMEM((CHUNK, d), jnp.float32),
            pltpu.VMEM_SHARED((n, d), jnp.float32),
            pltpu.SemaphoreType.DMA,
            pltpu.SemaphoreType.DMA,
            pltpu.SemaphoreType.DMA,
            pltpu.SemaphoreType.DMA,
        ],
    )
    def seg_sum(x_hbm, src_hbm, dst_hbm, zeros_hbm, out_hbm,
                sidx_all, didx0, didx1, rows0, rows1, acc,
                dsem0, dsem1, gsem0, gsem1):
        cid = lax.axis_index("c")
        sid = lax.axis_index("s")
        wid = sid * NC + cid
        base = wid * EDGES_PER_WORKER

        didx = (didx0, didx1)
        rows = (rows0, rows1)
        dsem = (dsem0, dsem1)
        gsem = (gsem0, gsem1)

        def start_chunk(i, b):
            pltpu.make_async_copy(
                dst_hbm.at[pl.ds(base + i * CHUNK, CHUNK)], didx[b], dsem[b]
            ).start()
            pltpu.make_async_copy(
                x_hbm.at[sidx_all.at[pl.ds(i * CHUNK, CHUNK)]], rows[b], gsem[b]
            ).start()

        def finish_chunk(b):
            pltpu.make_async_copy(
                dst_hbm.at[pl.ds(base, CHUNK)], didx[b], dsem[b]
            ).wait()
            pltpu.make_async_copy(
                x_hbm.at[pl.ds(0, CHUNK)], rows[b], gsem[b]
            ).wait()
            pltpu.sync_copy(rows[b], acc.at[didx[b]], add=True)

        @pl.when(sid < NS - 1)
        def _():
            pltpu.sync_copy(zeros_hbm.at[pl.ds(sid * STRIPE, STRIPE)],
                            acc.at[pl.ds(sid * STRIPE, STRIPE)])

        @pl.when(sid == NS - 1)
        def _():
            pltpu.sync_copy(zeros_hbm.at[pl.ds((NS - 1) * STRIPE,
                                               n - (NS - 1) * STRIPE)],
                            acc.at[pl.ds((NS - 1) * STRIPE,
                                         n - (NS - 1) * STRIPE)])

        pltpu.sync_copy(src_hbm.at[pl.ds(base, EDGES_PER_WORKER)], sidx_all)
        plsc.subcore_barrier()

        start_chunk(0, 0)
        start_chunk(1, 1)

        @pl.loop(0, NCHUNKS - 1, step=2)
        def _(i):
            finish_chunk(0)

            @pl.when(i + 2 < NCHUNKS)
            def _():
                start_chunk(i + 2, 0)

            finish_chunk(1)

            @pl.when(i + 3 < NCHUNKS)
            def _():
                start_chunk(i + 3, 1)

        finish_chunk(0)

        plsc.subcore_barrier()

        @pl.when(sid < NS - 1)
        def _():
            pltpu.sync_copy(acc.at[pl.ds(sid * STRIPE, STRIPE)],
                            out_hbm.at[cid, pl.ds(sid * STRIPE, STRIPE)])

        @pl.when(sid == NS - 1)
        def _():
            pltpu.sync_copy(acc.at[pl.ds((NS - 1) * STRIPE,
                                         n - (NS - 1) * STRIPE)],
                            out_hbm.at[cid, pl.ds((NS - 1) * STRIPE,
                                                  n - (NS - 1) * STRIPE)])

    return seg_sum(x, src, dst, zeros)


def _mlp1_tc(x, p0, p1, W1, b1, W2, b2, eps0):
    n, d_in = x.shape
    h_dim = W1.shape[1]
    blk = 2000

    def body(eps_ref, x_ref, p0_ref, p1_ref, w1_ref, b1_ref, w2_ref, b2_ref,
             o_ref):
        t = (1.0 + eps_ref[0]) * x_ref[...] + p0_ref[...] + p1_ref[...]
        h = jnp.dot(t, w1_ref[...], preferred_element_type=jnp.float32,
                    precision=lax.Precision.HIGHEST) + b1_ref[...]
        h = jnp.maximum(h, 0.0)
        h = jnp.dot(h, w2_ref[...], preferred_element_type=jnp.float32,
                    precision=lax.Precision.HIGHEST) + b2_ref[...]
        h = jnp.maximum(h, 0.0)
        o_ref[...] = jnp.concatenate([h, jnp.zeros_like(h)], axis=1)

    grid = (n // blk,)
    row_spec = pl.BlockSpec((blk, d_in), lambda i: (i, 0))
    return pl.pallas_call(
        body,
        grid=grid,
        in_specs=[
            pl.BlockSpec(memory_space=pltpu.SMEM),
            row_spec, row_spec, row_spec,
            pl.BlockSpec((d_in, h_dim), lambda i: (0, 0)),
            pl.BlockSpec((1, h_dim), lambda i: (0, 0)),
            pl.BlockSpec((h_dim, h_dim), lambda i: (0, 0)),
            pl.BlockSpec((1, h_dim), lambda i: (0, 0)),
        ],
        out_specs=pl.BlockSpec((blk, 2 * h_dim), lambda i: (i, 0)),
        out_shape=jax.ShapeDtypeStruct((n, 2 * h_dim), jnp.float32),
    )(eps0.reshape(1), x, p0, p1, W1, b1.reshape(1, -1), W2, b2.reshape(1, -1))


def _mlp2_tc(h, p0, p1, W3, b3, W4, b4, eps1):
    n, pad_dim = h.shape
    h_dim = W3.shape[0]
    d_out = W4.shape[1]
    blk = 2000

    def body(eps_ref, h_ref, p0_ref, p1_ref, w3_ref, b3_ref, w4_ref, b4_ref,
             o_ref):
        t = (1.0 + eps_ref[0]) * h_ref[...] + p0_ref[...] + p1_ref[...]
        t = t[:, :h_dim]
        g = jnp.dot(t, w3_ref[...], preferred_element_type=jnp.float32,
                    precision=lax.Precision.HIGHEST) + b3_ref[...]
        g = jnp.maximum(g, 0.0)
        logits = jnp.dot(g, w4_ref[...], preferred_element_type=jnp.float32,
                         precision=lax.Precision.HIGHEST) + b4_ref[...]
        m = jnp.max(logits, axis=1, keepdims=True)
        z = logits - m
        lse = jnp.log(jnp.sum(jnp.exp(z), axis=1, keepdims=True))
        o_ref[...] = z - lse

    grid = (n // blk,)
    row_spec = pl.BlockSpec((blk, pad_dim), lambda i: (i, 0))
    return pl.pallas_call(
        body,
        grid=grid,
        in_specs=[
            pl.BlockSpec(memory_space=pltpu.SMEM),
            row_spec, row_spec, row_spec,
            pl.BlockSpec((h_dim, h_dim), lambda i: (0, 0)),
            pl.BlockSpec((1, h_dim), lambda i: (0, 0)),
            pl.BlockSpec((h_dim, d_out), lambda i: (0, 0)),
            pl.BlockSpec((1, d_out), lambda i: (0, 0)),
        ],
        out_specs=pl.BlockSpec((blk, d_out), lambda i: (i, 0)),
        out_shape=jax.ShapeDtypeStruct((n, d_out), jnp.float32),
    )(eps1.reshape(1), h, p0, p1, W3, b3.reshape(1, -1), W4, b4.reshape(1, -1))


def kernel(x, edge_index, W1, b1, W2, b2, eps0, W3, b3, W4, b4, eps1):
    src = edge_index[0]
    dst = edge_index[1]

    parts = _segment_sum_sc(x, src, dst)
    h = _mlp1_tc(x, parts[0], parts[1], W1, b1, W2, b2, eps0)

    parts2 = _segment_sum_sc(h, src, dst)
    return _mlp2_tc(h, parts2[0], parts2[1], W3, b3, W4, b4, eps1)
